# baseline (speedup 1.0000x reference)
import jax
import jax.numpy as jnp
from jax import lax
from jax.experimental import pallas as pl
from jax.experimental.pallas import tpu as pltpu
from jax.experimental.pallas import tpu_sc as plsc

_C = 14
_MULT_UNLABELED = 3.0
_EPS = 1e-06
_LN2 = 0.6931471805599453
_LANE = 128

_SC_ROWS = 2112
_TC_ROWS = 4800
_TM = 800

_NW = 32
_V = _SC_ROWS * _LANE // 16
_NVEC = _V // 16

_LOG2_POLY = (
    -0.025123260071615975,
    0.11929850256851415,
    -0.27462368908179047,
    0.45552740717188667,
    -0.7175579830638209,
    1.4424753308482419,
    2.1230901012803116e-06,
)


def _tc_body(annot_ref, probs_a, probs_b, target_a, target_b, out_ref,
             ent_acc, ce_acc, fg_acc):
    b = pl.program_id(0)
    j = pl.program_id(1)
    nj = pl.num_programs(1)

    @pl.when(j == 0)
    def _init():
        ent_acc[...] = jnp.zeros_like(ent_acc)
        ce_acc[...] = jnp.zeros_like(ce_acc)
        fg_acc[0] = 0

    un = []
    for c in range(1, _C):
        pres = annot_ref[b, 0] == c
        for k in range(1, annot_ref.shape[1]):
            pres = pres | (annot_ref[b, k] == c)
        un.append(jnp.where(pres, 0.0, 1.0))

    ent_t = jnp.zeros((8, _LANE), jnp.float32)
    ce_t = jnp.zeros((8, _LANE), jnp.float32)
    fg_m = None
    for probs_ref, target_ref in ((probs_a, target_a), (probs_b, target_b)):
        for g in range(_TM // 8):
            sl = slice(g * 8, g * 8 + 8)
            t_v = target_ref[0, sl, :]
            p0 = probs_ref[0, 0, sl, :]
            ent_g = p0 * jnp.log(jnp.clip(p0, _EPS, 1.0 - _EPS))
            qfg = p0
            sun = p0
            for c in range(1, _C):
                p_c = probs_ref[0, c, sl, :]
                ent_g = ent_g + p_c * jnp.log(jnp.clip(p_c, _EPS, 1.0 - _EPS))
                qfg = jnp.where(t_v == c, p_c, qfg)
                sun = sun + p_c * un[c - 1]
            q = jnp.where(t_v == 0, sun, qfg)
            omq = 1.0 - q
            ce_t = ce_t - (omq * omq) * jnp.log(jnp.clip(q, _EPS, 1.0 - _EPS))
            ent_t = ent_t + ent_g
        tm = jnp.max(target_ref[0])
        fg_m = tm if fg_m is None else jnp.maximum(fg_m, tm)

    ent_acc[...] += ent_t
    ce_acc[...] += ce_t
    fg_acc[0] = jnp.maximum(fg_acc[0], fg_m)

    @pl.when(j == nj - 1)
    def _fini():
        out_ref[b, 0] = jnp.sum(ent_acc[...])
        out_ref[b, 1] = jnp.sum(ce_acc[...])
        out_ref[b, 2] = fg_acc[0].astype(jnp.float32)


def _tc_call(p4, t3, annot):
    B, C = p4.shape[0], p4.shape[1]
    nj = _TC_ROWS // (2 * _TM)
    return pl.pallas_call(
        _tc_body,
        grid=(B, nj),
        in_specs=[
            pl.BlockSpec(memory_space=pltpu.SMEM),
            pl.BlockSpec((1, C, _TM, _LANE), lambda b, j: (b, 0, 2 * j, 0)),
            pl.BlockSpec((1, C, _TM, _LANE), lambda b, j: (b, 0, 2 * j + 1, 0)),
            pl.BlockSpec((1, _TM, _LANE), lambda b, j: (b, 2 * j, 0)),
            pl.BlockSpec((1, _TM, _LANE), lambda b, j: (b, 2 * j + 1, 0)),
        ],
        out_specs=pl.BlockSpec(memory_space=pltpu.SMEM),
        out_shape=jax.ShapeDtypeStruct((B, 3), jnp.float32),
        scratch_shapes=[
            pltpu.VMEM((8, _LANE), jnp.float32),
            pltpu.VMEM((8, _LANE), jnp.float32),
            pltpu.SMEM((1,), jnp.int32),
        ],
    )(annot, p4, p4, t3, t3)


def _log2_clip(x):
    x = jnp.minimum(jnp.maximum(x, _EPS), 1.0 - _EPS)
    bits = lax.bitcast_convert_type(x, jnp.int32)
    e = (bits >> 23) - 127
    m = lax.bitcast_convert_type(
        (bits & 0x007FFFFF) | 0x3F800000, jnp.float32)
    t = m - 1.0
    acc = jnp.full(x.shape, _LOG2_POLY[0], jnp.float32)
    for co in _LOG2_POLY[1:]:
        acc = acc * t + co
    return e.astype(jnp.float32) + acc


def _sc_body(n_vox, p_hbm, t_hbm, un_hbm, out_hbm,
             in0, in1, qfg_v, sun_v, tbuf, unbuf, obuf,
             sem0, sem1, semt):
    cid = lax.axis_index("c")
    sid = lax.axis_index("s")
    wid = sid * 2 + cid
    b = wid // 16
    wr = wid % 16
    v0 = _TC_ROWS * _LANE + wr * _V

    ct = pltpu.make_async_copy(
        t_hbm.at[pl.ds(b * n_vox + v0, _V)], tbuf, semt)
    ct.start()

    bufs = (in0, in1)
    sems = (sem0, sem1)

    def fire(c):
        pltpu.make_async_copy(
            p_hbm.at[pl.ds((b * _C + c) * n_vox + v0, _V)],
            bufs[c % 2], sems[c % 2]).start()

    def drain(c):
        pltpu.make_async_copy(
            p_hbm.at[pl.ds((b * _C + c) * n_vox + v0, _V)],
            bufs[c % 2], sems[c % 2]).wait()

    fire(0)
    fire(1)

    pltpu.sync_copy(un_hbm.at[pl.ds(b * _C * 16, _C * 16)], unbuf)
    un_vecs = [unbuf[pl.ds(c * 16, 16)] for c in range(1, _C)]

    ct.wait()

    ent = jnp.zeros((16,), jnp.float32)
    for c in range(_C):
        drain(c)
        pbuf = bufs[c % 2]

        if c == 0:
            @plsc.parallel_loop(0, _NVEC, unroll=8, carry=ent)
            def class0_body(i, ent, pbuf=pbuf):
                off = i * 16
                pc = pbuf[pl.ds(off, 16)]
                ent = ent + pc * _log2_clip(pc)
                qfg_v[pl.ds(off, 16)] = pc
                sun_v[pl.ds(off, 16)] = pc
                return ent
            ent = class0_body
        else:
            @plsc.parallel_loop(0, _NVEC, unroll=8, carry=ent)
            def classc_body(i, ent, c=c, un_c=un_vecs[c - 1], pbuf=pbuf):
                off = i * 16
                pc = pbuf[pl.ds(off, 16)]
                tv = tbuf[pl.ds(off, 16)]
                ent = ent + pc * _log2_clip(pc)
                qfg_v[pl.ds(off, 16)] = jnp.where(
                    tv == c, pc, qfg_v[pl.ds(off, 16)])
                sun_v[pl.ds(off, 16)] = sun_v[pl.ds(off, 16)] + pc * un_c
                return ent
            ent = classc_body

        if c + 2 < _C:
            fire(c + 2)

    zero = jnp.zeros((16,), jnp.float32)
    fg0 = jnp.zeros((16,), jnp.int32)

    @plsc.parallel_loop(0, _NVEC, unroll=8, carry=(zero, fg0))
    def fini_body(i, carry):
        ce, fg = carry
        off = i * 16
        tv = tbuf[pl.ds(off, 16)]
        q = jnp.where(tv == 0, sun_v[pl.ds(off, 16)], qfg_v[pl.ds(off, 16)])
        omq = 1.0 - q
        ce = ce + (omq * omq) * _log2_clip(q)
        fg = jnp.maximum(fg, tv)
        return ce, fg

    ce, fg = fini_body

    obuf[pl.ds(0, 16)] = ent
    pltpu.sync_copy(obuf, out_hbm.at[0, wid])
    obuf[pl.ds(0, 16)] = ce
    pltpu.sync_copy(obuf, out_hbm.at[1, wid])
    obuf[pl.ds(0, 16)] = fg.astype(jnp.float32)
    pltpu.sync_copy(obuf, out_hbm.at[2, wid])


def _sc_call(p3, t2, un16, n_vox):
    import functools as _ft
    C = _C
    mesh = plsc.VectorSubcoreMesh(core_axis_name="c", subcore_axis_name="s")
    sc_fn = pl.kernel(
        _ft.partial(_sc_body, n_vox),
        out_type=jax.ShapeDtypeStruct((3, _NW, 16), jnp.float32),
        mesh=mesh,
        scratch_types=[
            pltpu.VMEM((_V,), jnp.float32),
            pltpu.VMEM((_V,), jnp.float32),
            pltpu.VMEM((_V,), jnp.float32),
            pltpu.VMEM((_V,), jnp.float32),
            pltpu.VMEM((_V,), jnp.int32),
            pltpu.VMEM((C * 16,), jnp.float32),
            pltpu.VMEM((16,), jnp.float32),
            pltpu.SemaphoreType.DMA,
            pltpu.SemaphoreType.DMA,
            pltpu.SemaphoreType.DMA,
        ],
    )
    return sc_fn(p3, t2, un16)


def kernel(probs, target, annotated_fg_categories):
    B, C = probs.shape[0], probs.shape[1]
    n_vox = probs.shape[2] * probs.shape[3] * probs.shape[4]
    rows = n_vox // _LANE

    p4 = probs.reshape(B, C, rows, _LANE)
    t3 = target.reshape(B, rows, _LANE)
    p3 = probs.reshape(B, C, n_vox)
    t2 = target.reshape(B, n_vox)

    ks = jnp.arange(C)
    annot = annotated_fg_categories
    present = jnp.any(
        (annot[:, None, :] == ks[None, :, None]) & (annot[:, None, :] > 0),
        axis=2)
    un = jnp.where(present, 0.0, 1.0).astype(jnp.float32)
    un16 = jnp.broadcast_to(un[:, :, None], (B, C, 16))

    sc_out = _sc_call(p3.reshape(-1), t2.reshape(-1), un16.reshape(-1),
                      n_vox)
    tc_out = _tc_call(p4, t3, annot)

    b_of_w = jnp.arange(_NW) // 16
    sc_ent = jnp.zeros((B,), jnp.float32).at[b_of_w].add(
        jnp.sum(sc_out[0], axis=1)) * _LN2
    sc_ce = jnp.zeros((B,), jnp.float32).at[b_of_w].add(
        jnp.sum(sc_out[1], axis=1)) * (-_LN2)
    sc_fg = jnp.zeros((B,), jnp.float32).at[b_of_w].max(
        jnp.max(sc_out[2], axis=1))

    ent_b = tc_out[:, 0] + sc_ent
    ce_b = tc_out[:, 1] + sc_ce
    fg_b = jnp.maximum(tc_out[:, 2], sc_fg)

    nf = jnp.float32(n_vox)
    mult = jnp.where(fg_b > 0.0, 1.0, _MULT_UNLABELED)
    reg = -jnp.sum(mult * (ent_b / nf)) / B
    ce = jnp.mean(ce_b / nf)
    return ce, reg


# speedup vs baseline: 1.0042x; 1.0042x over previous
import jax
import jax.numpy as jnp
from jax import lax
from jax.experimental import pallas as pl
from jax.experimental.pallas import tpu as pltpu
from jax.experimental.pallas import tpu_sc as plsc

_C = 14
_MULT_UNLABELED = 3.0
_EPS = 1e-06
_LN2 = 0.6931471805599453
_LANE = 128

_SC_ROWS = 2112
_TC_ROWS = 4800
_TM = 800

_NW = 32
_V = _SC_ROWS * _LANE // 16
_NVEC = _V // 16

_LOG2_POLY = (
    -0.025123260071615975,
    0.11929850256851415,
    -0.27462368908179047,
    0.45552740717188667,
    -0.7175579830638209,
    1.4424753308482419,
    2.1230901012803116e-06,
)


def _tc_body(annot_ref, probs_a, probs_b, target_a, target_b, out_ref,
             ent_acc, ce_acc, fg_acc):
    b = pl.program_id(0)
    j = pl.program_id(1)
    nj = pl.num_programs(1)

    @pl.when(j == 0)
    def _init():
        ent_acc[...] = jnp.zeros_like(ent_acc)
        ce_acc[...] = jnp.zeros_like(ce_acc)
        fg_acc[0] = 0

    un = []
    for c in range(1, _C):
        pres = annot_ref[b, 0] == c
        for k in range(1, annot_ref.shape[1]):
            pres = pres | (annot_ref[b, k] == c)
        un.append(jnp.where(pres, 0.0, 1.0))

    ent_t = jnp.zeros((8, _LANE), jnp.float32)
    ce_t = jnp.zeros((8, _LANE), jnp.float32)
    fg_m = None
    for probs_ref, target_ref in ((probs_a, target_a), (probs_b, target_b)):
        for g in range(_TM // 8):
            sl = slice(g * 8, g * 8 + 8)
            t_v = target_ref[0, sl, :]
            p0 = probs_ref[0, 0, sl, :]
            ent_g = p0 * jnp.log(jnp.clip(p0, _EPS, 1.0 - _EPS))
            qfg = p0
            sun = p0
            for c in range(1, _C):
                p_c = probs_ref[0, c, sl, :]
                ent_g = ent_g + p_c * jnp.log(jnp.clip(p_c, _EPS, 1.0 - _EPS))
                qfg = jnp.where(t_v == c, p_c, qfg)
                sun = sun + p_c * un[c - 1]
            q = jnp.where(t_v == 0, sun, qfg)
            omq = 1.0 - q
            ce_t = ce_t - (omq * omq) * jnp.log(jnp.clip(q, _EPS, 1.0 - _EPS))
            ent_t = ent_t + ent_g
        tm = jnp.max(target_ref[0])
        fg_m = tm if fg_m is None else jnp.maximum(fg_m, tm)

    ent_acc[...] += ent_t
    ce_acc[...] += ce_t
    fg_acc[0] = jnp.maximum(fg_acc[0], fg_m)

    @pl.when(j == nj - 1)
    def _fini():
        out_ref[b, 0] = jnp.sum(ent_acc[...])
        out_ref[b, 1] = jnp.sum(ce_acc[...])
        out_ref[b, 2] = fg_acc[0].astype(jnp.float32)


def _tc_call(p4, t3, annot):
    B, C = p4.shape[0], p4.shape[1]
    nj = _TC_ROWS // (2 * _TM)
    return pl.pallas_call(
        _tc_body,
        grid=(B, nj),
        in_specs=[
            pl.BlockSpec(memory_space=pltpu.SMEM),
            pl.BlockSpec((1, C, _TM, _LANE), lambda b, j: (b, 0, 2 * j, 0)),
            pl.BlockSpec((1, C, _TM, _LANE), lambda b, j: (b, 0, 2 * j + 1, 0)),
            pl.BlockSpec((1, _TM, _LANE), lambda b, j: (b, 2 * j, 0)),
            pl.BlockSpec((1, _TM, _LANE), lambda b, j: (b, 2 * j + 1, 0)),
        ],
        out_specs=pl.BlockSpec(memory_space=pltpu.SMEM),
        out_shape=jax.ShapeDtypeStruct((B, 3), jnp.float32),
        scratch_shapes=[
            pltpu.VMEM((8, _LANE), jnp.float32),
            pltpu.VMEM((8, _LANE), jnp.float32),
            pltpu.SMEM((1,), jnp.int32),
        ],
    )(annot, p4, p4, t3, t3)


def _log2_clip(x):
    x = jnp.minimum(jnp.maximum(x, _EPS), 1.0 - _EPS)
    bits = lax.bitcast_convert_type(x, jnp.int32)
    e = (bits >> 23) - 127
    m = lax.bitcast_convert_type(
        (bits & 0x007FFFFF) | 0x3F800000, jnp.float32)
    t = m - 1.0
    acc = jnp.full(x.shape, _LOG2_POLY[0], jnp.float32)
    for co in _LOG2_POLY[1:]:
        acc = acc * t + co
    return e.astype(jnp.float32) + acc


def _sc_body(n_vox, p_hbm, t_hbm, un_hbm, out_hbm,
             in0, in1, in2, in3, qfg_v, sun_v, tbuf, unbuf, obuf,
             sem0, sem1, sem2, sem3, semt):
    cid = lax.axis_index("c")
    sid = lax.axis_index("s")
    wid = sid * 2 + cid
    b = wid // 16
    wr = wid % 16
    v0 = _TC_ROWS * _LANE + wr * _V

    ct = pltpu.make_async_copy(
        t_hbm.at[pl.ds(b * n_vox + v0, _V)], tbuf, semt)
    ct.start()

    bufs = (in0, in1, in2, in3)
    sems = (sem0, sem1, sem2, sem3)

    def fire(c):
        pltpu.make_async_copy(
            p_hbm.at[pl.ds((b * _C + c) * n_vox + v0, _V)],
            bufs[c % 4], sems[c % 4]).start()

    def drain(c):
        pltpu.make_async_copy(
            p_hbm.at[pl.ds((b * _C + c) * n_vox + v0, _V)],
            bufs[c % 4], sems[c % 4]).wait()

    fire(0)
    fire(1)
    fire(2)
    fire(3)

    pltpu.sync_copy(un_hbm.at[pl.ds(b * _C * 16, _C * 16)], unbuf)
    un_vecs = [unbuf[pl.ds(c * 16, 16)] for c in range(1, _C)]

    ct.wait()

    ent = jnp.zeros((16,), jnp.float32)
    for c in range(_C):
        drain(c)
        pbuf = bufs[c % 4]

        if c == 0:
            @plsc.parallel_loop(0, _NVEC, unroll=8, carry=ent)
            def class0_body(i, ent, pbuf=pbuf):
                off = i * 16
                pc = pbuf[pl.ds(off, 16)]
                ent = ent + pc * _log2_clip(pc)
                qfg_v[pl.ds(off, 16)] = pc
                sun_v[pl.ds(off, 16)] = pc
                return ent
            ent = class0_body
        else:
            @plsc.parallel_loop(0, _NVEC, unroll=8, carry=ent)
            def classc_body(i, ent, c=c, un_c=un_vecs[c - 1], pbuf=pbuf):
                off = i * 16
                pc = pbuf[pl.ds(off, 16)]
                tv = tbuf[pl.ds(off, 16)]
                ent = ent + pc * _log2_clip(pc)
                qfg_v[pl.ds(off, 16)] = jnp.where(
                    tv == c, pc, qfg_v[pl.ds(off, 16)])
                sun_v[pl.ds(off, 16)] = sun_v[pl.ds(off, 16)] + pc * un_c
                return ent
            ent = classc_body

        if c + 4 < _C:
            fire(c + 4)

    zero = jnp.zeros((16,), jnp.float32)
    fg0 = jnp.zeros((16,), jnp.int32)

    @plsc.parallel_loop(0, _NVEC, unroll=8, carry=(zero, fg0))
    def fini_body(i, carry):
        ce, fg = carry
        off = i * 16
        tv = tbuf[pl.ds(off, 16)]
        q = jnp.where(tv == 0, sun_v[pl.ds(off, 16)], qfg_v[pl.ds(off, 16)])
        omq = 1.0 - q
        ce = ce + (omq * omq) * _log2_clip(q)
        fg = jnp.maximum(fg, tv)
        return ce, fg

    ce, fg = fini_body

    obuf[pl.ds(0, 16)] = ent
    pltpu.sync_copy(obuf, out_hbm.at[0, wid])
    obuf[pl.ds(0, 16)] = ce
    pltpu.sync_copy(obuf, out_hbm.at[1, wid])
    obuf[pl.ds(0, 16)] = fg.astype(jnp.float32)
    pltpu.sync_copy(obuf, out_hbm.at[2, wid])


def _sc_call(p3, t2, un16, n_vox):
    import functools as _ft
    C = _C
    mesh = plsc.VectorSubcoreMesh(core_axis_name="c", subcore_axis_name="s")
    sc_fn = pl.kernel(
        _ft.partial(_sc_body, n_vox),
        out_type=jax.ShapeDtypeStruct((3, _NW, 16), jnp.float32),
        mesh=mesh,
        scratch_types=[
            pltpu.VMEM((_V,), jnp.float32),
            pltpu.VMEM((_V,), jnp.float32),
            pltpu.VMEM((_V,), jnp.float32),
            pltpu.VMEM((_V,), jnp.float32),
            pltpu.VMEM((_V,), jnp.float32),
            pltpu.VMEM((_V,), jnp.float32),
            pltpu.VMEM((_V,), jnp.int32),
            pltpu.VMEM((C * 16,), jnp.float32),
            pltpu.VMEM((16,), jnp.float32),
            pltpu.SemaphoreType.DMA,
            pltpu.SemaphoreType.DMA,
            pltpu.SemaphoreType.DMA,
            pltpu.SemaphoreType.DMA,
            pltpu.SemaphoreType.DMA,
        ],
    )
    return sc_fn(p3, t2, un16)


def kernel(probs, target, annotated_fg_categories):
    B, C = probs.shape[0], probs.shape[1]
    n_vox = probs.shape[2] * probs.shape[3] * probs.shape[4]
    rows = n_vox // _LANE

    p4 = probs.reshape(B, C, rows, _LANE)
    t3 = target.reshape(B, rows, _LANE)
    p3 = probs.reshape(B, C, n_vox)
    t2 = target.reshape(B, n_vox)

    ks = jnp.arange(C)
    annot = annotated_fg_categories
    present = jnp.any(
        (annot[:, None, :] == ks[None, :, None]) & (annot[:, None, :] > 0),
        axis=2)
    un = jnp.where(present, 0.0, 1.0).astype(jnp.float32)
    un16 = jnp.broadcast_to(un[:, :, None], (B, C, 16))

    sc_out = _sc_call(p3.reshape(-1), t2.reshape(-1), un16.reshape(-1),
                      n_vox)
    tc_out = _tc_call(p4, t3, annot)

    b_of_w = jnp.arange(_NW) // 16
    sc_ent = jnp.zeros((B,), jnp.float32).at[b_of_w].add(
        jnp.sum(sc_out[0], axis=1)) * _LN2
    sc_ce = jnp.zeros((B,), jnp.float32).at[b_of_w].add(
        jnp.sum(sc_out[1], axis=1)) * (-_LN2)
    sc_fg = jnp.zeros((B,), jnp.float32).at[b_of_w].max(
        jnp.max(sc_out[2], axis=1))

    ent_b = tc_out[:, 0] + sc_ent
    ce_b = tc_out[:, 1] + sc_ce
    fg_b = jnp.maximum(tc_out[:, 2], sc_fg)

    nf = jnp.float32(n_vox)
    mult = jnp.where(fg_b > 0.0, 1.0, _MULT_UNLABELED)
    reg = -jnp.sum(mult * (ent_b / nf)) / B
    ce = jnp.mean(ce_b / nf)
    return ce, reg
